# baseline (speedup 1.0000x reference)
import functools
import jax
import jax.numpy as jnp
from jax import lax
from jax.experimental import pallas as pl
from jax.experimental.pallas import tpu as pltpu
from jax.experimental.pallas import tpu_sc as plsc

_NC = 2
_NS = 16
_NW = _NC * _NS
_L = 16
_CHUNK = 32
_NBUF = 2


def kernel(token_embeddings, pos_table):
    batch, seq, dim = token_embeddings.shape
    tok_flat = token_embeddings.reshape(batch * seq * dim)
    pos_flat = pos_table.reshape(seq * dim)

    seq_per_w = seq // _NW
    steps = seq_per_w // _CHUNK
    chunk_words = _CHUNK * dim
    n_items = steps * batch

    mesh = plsc.VectorSubcoreMesh(
        core_axis_name="c", subcore_axis_name="s",
        num_cores=_NC, num_subcores=_NS,
    )

    @functools.partial(
        pl.kernel,
        out_type=jax.ShapeDtypeStruct((batch * seq * dim,), jnp.float32),
        mesh=mesh,
        scratch_types=[
            pltpu.VMEM((chunk_words,), jnp.float32),
            pltpu.VMEM((_NBUF, chunk_words), jnp.float32),
            pltpu.SemaphoreType.DMA,
            pltpu.SemaphoreType.DMA,
            pltpu.SemaphoreType.DMA,
            pltpu.SemaphoreType.DMA,
            pltpu.SemaphoreType.DMA,
        ],
    )
    def sc_add(tok_hbm, pos_hbm, out_hbm, pbuf, obuf, psem,
               isem0, isem1, osem0, osem1):
        w = lax.axis_index("s") * _NC + lax.axis_index("c")
        seq0 = w * seq_per_w
        isems = (isem0, isem1)
        osems = (osem0, osem1)

        def hbm_off(k):
            t, b = divmod(k, batch)
            return (b * seq + seq0 + t * _CHUNK) * dim

        def start_in(k):
            return pltpu.async_copy(
                tok_hbm.at[pl.ds(hbm_off(k), chunk_words)],
                obuf.at[k % _NBUF], isems[k % _NBUF])

        def start_pos(t):
            return pltpu.async_copy(
                pos_hbm.at[pl.ds((seq0 + t * _CHUNK) * dim, chunk_words)],
                pbuf, psem)

        pos_dma = {0: start_pos(0)}
        in_dma = {0: start_in(0), 1: start_in(1)}
        out_dma = {}

        for k in range(n_items):
            t, b = divmod(k, batch)
            buf = k % _NBUF
            if b == 0:
                pos_dma.pop(t).wait()
            in_dma.pop(k).wait()

            @plsc.parallel_loop(0, chunk_words, step=_L, unroll=8)
            def vbody(i):
                plsc.addupdate(obuf.at[buf, pl.ds(i, _L)],
                               pbuf[pl.ds(i, _L)])

            out_dma[k] = pltpu.async_copy(
                obuf.at[buf], out_hbm.at[pl.ds(hbm_off(k), chunk_words)],
                osems[buf])
            if b == batch - 1 and t + 1 < steps:
                pos_dma[t + 1] = start_pos(t + 1)
            if k + 2 < n_items:
                out_dma.pop(k).wait()
                in_dma[k + 2] = start_in(k + 2)

        for d in out_dma.values():
            d.wait()

    out = sc_add(tok_flat, pos_flat)
    return out.reshape(batch, seq, dim)


# speedup vs baseline: 4.9050x; 4.9050x over previous
import jax
import jax.numpy as jnp
from jax.experimental import pallas as pl

_BS = 512


def _add_body(tok_ref, pos_ref, out_ref):
    out_ref[...] = tok_ref[...] + pos_ref[...][None, :, :]


def kernel(token_embeddings, pos_table):
    batch, seq, dim = token_embeddings.shape
    return pl.pallas_call(
        _add_body,
        grid=(seq // _BS,),
        in_specs=[
            pl.BlockSpec((batch, _BS, dim), lambda s: (0, s, 0)),
            pl.BlockSpec((_BS, dim), lambda s: (s, 0)),
        ],
        out_specs=pl.BlockSpec((batch, _BS, dim), lambda s: (0, s, 0)),
        out_shape=jax.ShapeDtypeStruct((batch, seq, dim), token_embeddings.dtype),
    )(token_embeddings, pos_table)


# speedup vs baseline: 4.9495x; 1.0091x over previous
import jax
import jax.numpy as jnp
from jax.experimental import pallas as pl

_BS = 2048


def _add_body(tok_ref, pos_ref, out_ref):
    out_ref[...] = tok_ref[...] + pos_ref[...][None, :, :]


def kernel(token_embeddings, pos_table):
    batch, seq, dim = token_embeddings.shape
    return pl.pallas_call(
        _add_body,
        grid=(seq // _BS, batch),
        in_specs=[
            pl.BlockSpec((1, _BS, dim), lambda s, b: (b, s, 0)),
            pl.BlockSpec((_BS, dim), lambda s, b: (s, 0)),
        ],
        out_specs=pl.BlockSpec((1, _BS, dim), lambda s, b: (b, s, 0)),
        out_shape=jax.ShapeDtypeStruct((batch, seq, dim), token_embeddings.dtype),
    )(token_embeddings, pos_table)
